# Initial kernel scaffold; baseline (speedup 1.0000x reference)
import jax
import jax.numpy as jnp
from jax.experimental import pallas as pl


def kernel(x, W_enc, b_enc, W_dec, b_dec):
    raise NotImplementedError("write your pallas kernel here")


# speedup vs baseline: 15.1894x; 15.1894x over previous
import functools

import jax
import jax.numpy as jnp
from jax.experimental import pallas as pl
from jax.experimental.pallas import tpu as pltpu

K = 64


def _enc_kernel(x_ref, w_ref, b_ref, post_ref):
    pre = jax.lax.dot_general(
        x_ref[...], w_ref[...],
        dimension_numbers=(((1,), (1,)), ((), ())),
        preferred_element_type=jnp.float32,
    )
    post_ref[...] = jnp.maximum(pre + b_ref[...][None, :], 0.0)


def _select_kernel(post_ref, f_ref, *, k):
    post = post_ref[...]
    rows = post.shape[0]
    bits = jax.lax.bitcast_convert_type(post, jnp.int32)

    def body(i, p):
        b = 30 - i
        cand = jnp.bitwise_or(p, jnp.int32(1) << b)
        cnt = jnp.sum((bits >= cand).astype(jnp.int32), axis=1, keepdims=True)
        return jnp.where(cnt >= k, cand, p)

    thresh = jax.lax.fori_loop(0, 31, body, jnp.zeros((rows, 1), jnp.int32))
    f_ref[...] = jnp.where(bits >= thresh, post, 0.0)


def _dec_kernel(f_ref, w_ref, b_ref, out_ref):
    @pl.when(pl.program_id(0) == 0)
    def _():
        out_ref[...] = jnp.broadcast_to(b_ref[...][None, :], out_ref.shape)

    out_ref[...] += jax.lax.dot_general(
        f_ref[...], w_ref[...],
        dimension_numbers=(((1,), (1,)), ((), ())),
        preferred_element_type=jnp.float32,
    )


def kernel(x, W_enc, b_enc, W_dec, b_dec):
    batch, d_model = x.shape
    n_features = W_enc.shape[0]

    bf = min(1024, n_features)
    nfb = n_features // bf

    post = pl.pallas_call(
        _enc_kernel,
        grid=(nfb,),
        in_specs=[
            pl.BlockSpec((batch, d_model), lambda i: (0, 0)),
            pl.BlockSpec((bf, d_model), lambda i: (i, 0)),
            pl.BlockSpec((bf,), lambda i: (i,)),
        ],
        out_specs=pl.BlockSpec((batch, bf), lambda i: (0, i)),
        out_shape=jax.ShapeDtypeStruct((batch, n_features), jnp.float32),
        compiler_params=pltpu.CompilerParams(
            dimension_semantics=("arbitrary",),
        ),
    )(x, W_enc, b_enc)

    rb = min(128, batch)
    f = pl.pallas_call(
        functools.partial(_select_kernel, k=K),
        grid=(batch // rb,),
        in_specs=[pl.BlockSpec((rb, n_features), lambda i: (i, 0))],
        out_specs=pl.BlockSpec((rb, n_features), lambda i: (i, 0)),
        out_shape=jax.ShapeDtypeStruct((batch, n_features), jnp.float32),
        compiler_params=pltpu.CompilerParams(
            dimension_semantics=("parallel",),
        ),
    )(post)

    x_hat = pl.pallas_call(
        _dec_kernel,
        grid=(nfb,),
        in_specs=[
            pl.BlockSpec((batch, bf), lambda i: (0, i)),
            pl.BlockSpec((d_model, bf), lambda i: (0, i)),
            pl.BlockSpec((d_model,), lambda i: (0,)),
        ],
        out_specs=pl.BlockSpec((batch, d_model), lambda i: (0, 0)),
        out_shape=jax.ShapeDtypeStruct((batch, d_model), jnp.float32),
        compiler_params=pltpu.CompilerParams(
            dimension_semantics=("arbitrary",),
        ),
    )(f, W_dec, b_dec)

    return (x_hat, f)


# speedup vs baseline: 18.5492x; 1.2212x over previous
import functools

import jax
import jax.numpy as jnp
from jax.experimental import pallas as pl
from jax.experimental.pallas import tpu as pltpu

K = 64


def _enc_kernel(x_ref, w_ref, b_ref, post_ref):
    pre = jax.lax.dot_general(
        x_ref[...], w_ref[...],
        dimension_numbers=(((1,), (1,)), ((), ())),
        preferred_element_type=jnp.float32,
    )
    post_ref[...] = jnp.maximum(pre + b_ref[...][None, :], 0.0)


def _select_kernel(post_ref, f_ref, lo_ref, hi_ref, t_ref, *, k):
    post = post_ref[...]
    rows, n = post.shape
    bits = jax.lax.bitcast_convert_type(post, jnp.int32)

    ngroups = n // 128

    def gmax_body(j, cm):
        return jnp.maximum(
            cm,
            jax.lax.bitcast_convert_type(
                post_ref[:, pl.ds(j * 128, 128)], jnp.int32),
        )

    cm = jax.lax.fori_loop(
        1, ngroups, gmax_body,
        jax.lax.bitcast_convert_type(post_ref[:, pl.ds(0, 128)], jnp.int32))

    def lb_body(i, p):
        cand = jnp.bitwise_or(p, jnp.int32(1) << (30 - i))
        cnt = jnp.sum((cm >= cand).astype(jnp.int32), axis=1, keepdims=True)
        return jnp.where(cnt >= k, cand, p)

    lo0 = jax.lax.fori_loop(0, 31, lb_body, jnp.zeros((rows, 1), jnp.int32))
    hi0 = jnp.max(cm, axis=1, keepdims=True) + 1

    lo_ref[...] = lo0
    hi_ref[...] = hi0
    t_ref[...] = jnp.full((rows, 1), -1, jnp.int32)

    def cond(state):
        it, n_act = state
        return jnp.logical_and(it < 34, n_act > 0)

    def body(state):
        it, _ = state
        lo = lo_ref[...]
        hi = hi_ref[...]
        t = t_ref[...]
        active = (hi - lo > 1) & (t < 0)
        mid = lo + ((hi - lo) >> 1)
        cnt = jnp.sum((bits >= mid).astype(jnp.int32), axis=1, keepdims=True)
        hit = (cnt == k) & active
        ge = cnt >= k
        t = jnp.where(hit, mid, t)
        lo = jnp.where(active & ge & (~hit), mid, lo)
        hi = jnp.where(active & (~ge), mid, hi)
        t_ref[...] = t
        lo_ref[...] = lo
        hi_ref[...] = hi
        n_act = jnp.sum(((hi - lo > 1) & (t < 0)).astype(jnp.int32))
        return it + 1, n_act

    n0 = jnp.sum((hi0 - lo0 > 1).astype(jnp.int32))
    jax.lax.while_loop(cond, body, (jnp.int32(0), n0))
    thresh = jnp.where(t_ref[...] < 0, lo_ref[...], t_ref[...])
    f_ref[...] = jnp.where(bits >= thresh, post, 0.0)


def _dec_kernel(f_ref, w_ref, b_ref, out_ref):
    @pl.when(pl.program_id(0) == 0)
    def _():
        out_ref[...] = jnp.broadcast_to(b_ref[...][None, :], out_ref.shape)

    out_ref[...] += jax.lax.dot_general(
        f_ref[...], w_ref[...],
        dimension_numbers=(((1,), (1,)), ((), ())),
        preferred_element_type=jnp.float32,
    )


def kernel(x, W_enc, b_enc, W_dec, b_dec):
    batch, d_model = x.shape
    n_features = W_enc.shape[0]

    bf = min(1024, n_features)
    nfb = n_features // bf

    post = pl.pallas_call(
        _enc_kernel,
        grid=(nfb,),
        in_specs=[
            pl.BlockSpec((batch, d_model), lambda i: (0, 0)),
            pl.BlockSpec((bf, d_model), lambda i: (i, 0)),
            pl.BlockSpec((bf,), lambda i: (i,)),
        ],
        out_specs=pl.BlockSpec((batch, bf), lambda i: (0, i)),
        out_shape=jax.ShapeDtypeStruct((batch, n_features), jnp.float32),
        compiler_params=pltpu.CompilerParams(
            dimension_semantics=("arbitrary",),
        ),
    )(x, W_enc, b_enc)

    rb = min(128, batch)
    f = pl.pallas_call(
        functools.partial(_select_kernel, k=K),
        grid=(batch // rb,),
        in_specs=[pl.BlockSpec((rb, n_features), lambda i: (i, 0))],
        out_specs=pl.BlockSpec((rb, n_features), lambda i: (i, 0)),
        out_shape=jax.ShapeDtypeStruct((batch, n_features), jnp.float32),
        scratch_shapes=[
            pltpu.VMEM((rb, 1), jnp.int32),
            pltpu.VMEM((rb, 1), jnp.int32),
            pltpu.VMEM((rb, 1), jnp.int32),
        ],
        compiler_params=pltpu.CompilerParams(
            dimension_semantics=("parallel",),
        ),
    )(post)

    x_hat = pl.pallas_call(
        _dec_kernel,
        grid=(nfb,),
        in_specs=[
            pl.BlockSpec((batch, bf), lambda i: (0, i)),
            pl.BlockSpec((d_model, bf), lambda i: (0, i)),
            pl.BlockSpec((d_model,), lambda i: (0,)),
        ],
        out_specs=pl.BlockSpec((batch, d_model), lambda i: (0, 0)),
        out_shape=jax.ShapeDtypeStruct((batch, d_model), jnp.float32),
        compiler_params=pltpu.CompilerParams(
            dimension_semantics=("arbitrary",),
        ),
    )(f, W_dec, b_dec)

    return (x_hat, f)
